# Initial kernel scaffold; baseline (speedup 1.0000x reference)
import functools

import jax
import jax.numpy as jnp
from jax import lax
from jax.experimental import pallas as pl
from jax.experimental.pallas import tpu as pltpu
from jax.experimental.pallas import tpu_sc as plsc

_CH = 128


def _make_edge_agg(n_ent, d_in, n_rel, chunks_per_core):
  mesh = plsc.VectorSubcoreMesh(core_axis_name="c", subcore_axis_name="s")
  ns = 16
  base_chunks = chunks_per_core // ns
  rem = chunks_per_core - ns * base_chunks
  rows_per_tile = n_ent // ns
  nlane = d_in // 16

  @functools.partial(
      pl.kernel,
      mesh=mesh,
      out_type=[
          jax.ShapeDtypeStruct((n_ent, d_in), jnp.float32),
          jax.ShapeDtypeStruct((n_ent, d_in), jnp.float32),
      ],
      scratch_types=[
          pltpu.VMEM_SHARED((n_ent, d_in), jnp.float32),
          pltpu.VMEM((base_chunks + 1, _CH), jnp.int32),
          pltpu.VMEM((base_chunks + 1, _CH), jnp.int32),
          pltpu.VMEM((base_chunks + 1, _CH), jnp.int32),
          pltpu.VMEM((base_chunks + 1, _CH), jnp.float32),
          pltpu.VMEM((2, _CH, d_in), jnp.float32),
          pltpu.VMEM((2, _CH, d_in), jnp.float32),
          pltpu.SemaphoreType.DMA,
          pltpu.SemaphoreType.DMA,
      ],
  )
  def edge_agg(ent_hbm, rel_hbm, src_hbm, typ_hbm, dst_hbm, nrm_hbm, zero_hbm,
               out_in, out_out, shared, src_v, typ_v, dst_v, nrm_v,
               erow, rrow, sem_a, sem_b):
    c = lax.axis_index("c")
    s = lax.axis_index("s")
    rb = s * rows_per_tile
    pltpu.sync_copy(zero_hbm.at[pl.ds(rb, rows_per_tile)],
                    shared.at[pl.ds(rb, rows_per_tile)])
    extra = jnp.minimum(s, rem)
    start = c * chunks_per_core + s * base_chunks + extra
    nchunks = base_chunks + jnp.where(s < rem, 1, 0)
    pltpu.sync_copy(src_hbm.at[pl.ds(start, base_chunks + 1)], src_v)
    pltpu.sync_copy(typ_hbm.at[pl.ds(start, base_chunks + 1)], typ_v)
    pltpu.sync_copy(dst_hbm.at[pl.ds(start, base_chunks + 1)], dst_v)
    pltpu.sync_copy(nrm_hbm.at[pl.ds(start, base_chunks + 1)], nrm_v)
    plsc.subcore_barrier()

    def fire(i, slot):
      pltpu.async_copy(ent_hbm.at[src_v.at[i]], erow.at[slot], sem_a)
      pltpu.async_copy(rel_hbm.at[typ_v.at[i]], rrow.at[slot], sem_b)

    def drain(slot):
      pltpu.make_async_copy(ent_hbm.at[src_v.at[0]], erow.at[slot], sem_a).wait()
      pltpu.make_async_copy(rel_hbm.at[typ_v.at[0]], rrow.at[slot], sem_b).wait()

    fire(0, 0)

    def chunk_body(i, carry):
      slot = lax.rem(i, 2)
      nxt = lax.rem(i + 1, 2)

      @pl.when(i + 1 < nchunks)
      def _():
        fire(i + 1, nxt)

      drain(slot)

      def edge_body(e, carry2):
        n = nrm_v[i, e]
        for j in range(nlane):
          sl = pl.ds(j * 16, 16)
          erow[slot, e, sl] = (erow[slot, e, sl] - rrow[slot, e, sl]) * n
        return carry2

      lax.fori_loop(0, _CH, edge_body, 0)
      pltpu.sync_copy(erow.at[slot], shared.at[dst_v.at[i]], add=True)
      return carry

    lax.fori_loop(0, nchunks, chunk_body, 0)
    plsc.subcore_barrier()

    @pl.when(c == 0)
    def _():
      pltpu.sync_copy(shared.at[pl.ds(rb, rows_per_tile)],
                      out_in.at[pl.ds(rb, rows_per_tile)])

    @pl.when(c == 1)
    def _():
      pltpu.sync_copy(shared.at[pl.ds(rb, rows_per_tile)],
                      out_out.at[pl.ds(rb, rows_per_tile)])

  return edge_agg


def _make_pair_gather(d, batch):
  mesh = plsc.VectorSubcoreMesh(core_axis_name="c", subcore_axis_name="s")
  nw = 32
  per = batch // nw

  @functools.partial(
      pl.kernel,
      mesh=mesh,
      out_type=[
          jax.ShapeDtypeStruct((batch, d), jnp.float32),
          jax.ShapeDtypeStruct((batch, d), jnp.float32),
      ],
      scratch_types=[
          pltpu.VMEM((per,), jnp.int32),
          pltpu.VMEM((per,), jnp.int32),
          pltpu.VMEM((per, d), jnp.float32),
          pltpu.VMEM((per, d), jnp.float32),
          pltpu.SemaphoreType.DMA,
          pltpu.SemaphoreType.DMA,
      ],
  )
  def pair_gather(x_hbm, r_hbm, head_hbm, rela_hbm, out_x, out_r,
                  hidx, ridx, xrow, rrow, sem_a, sem_b):
    c = lax.axis_index("c")
    s = lax.axis_index("s")
    base = (s * 2 + c) * per
    pltpu.sync_copy(head_hbm.at[pl.ds(base, per)], hidx)
    pltpu.sync_copy(rela_hbm.at[pl.ds(base, per)], ridx)
    ga = pltpu.async_copy(x_hbm.at[hidx], xrow, sem_a)
    gb = pltpu.async_copy(r_hbm.at[ridx], rrow, sem_b)
    ga.wait()
    gb.wait()
    pltpu.sync_copy(xrow, out_x.at[pl.ds(base, per)])
    pltpu.sync_copy(rrow, out_r.at[pl.ds(base, per)])

  return pair_gather


def _xpre_body(agg_in_ref, agg_out_ref, ent_ref, in_w_ref, out_w_ref,
               loop_w_ref, loop_rel_ref, bias_ref, rel_ref, w_rel_ref,
               xpre_ref, stats_ref, r_ref, acc, *, nblk):
  i = pl.program_id(0)
  f32 = jnp.float32
  xp = jnp.dot(agg_in_ref[...], in_w_ref[...], preferred_element_type=f32)
  xp += jnp.dot(agg_out_ref[...], out_w_ref[...], preferred_element_type=f32)
  xp += jnp.dot(ent_ref[...] - loop_rel_ref[...], loop_w_ref[...],
                preferred_element_type=f32)
  xp = xp * (1.0 / 3.0) + bias_ref[...]
  xpre_ref[...] = xp

  @pl.when(i == 0)
  def _():
    acc[...] = jnp.zeros_like(acc)
    r_ref[...] = jnp.dot(rel_ref[...], w_rel_ref[...], preferred_element_type=f32)

  acc[0:1, :] += jnp.sum(xp, axis=0, keepdims=True)
  acc[1:2, :] += jnp.sum(xp * xp, axis=0, keepdims=True)

  @pl.when(i == nblk - 1)
  def _():
    stats_ref[...] = acc[...]


def _decoder_body(xh_ref, rh_ref, stats_ref, gamma_ref, beta_ref,
                  emb_ref, bent_ref, out_ref, obj, *, n_ent, cblk):
  i = pl.program_id(0)

  @pl.when(i == 0)
  def _():
    inv_n = 1.0 / n_ent
    mean = stats_ref[0:1, :] * inv_n
    var = stats_ref[1:2, :] * inv_n - mean * mean
    xn = (xh_ref[...] - mean) * lax.rsqrt(var + 1e-5)
    xn = jnp.tanh(xn * gamma_ref[...] + beta_ref[...])
    obj[...] = xn * rh_ref[...]

  logits = lax.dot_general(obj[...], emb_ref[...], (((1,), (1,)), ((), ())),
                           preferred_element_type=jnp.float32)
  logits += bent_ref[0:1, pl.ds(i * cblk, cblk)]
  out_ref[...] = jax.nn.sigmoid(logits)


def kernel(ent_emb, rel_emb, in_w, out_w, loop_w, w_rel, loop_rel, bias_cov,
           bn_gamma, bn_beta, b_ent, emb_ent, edge_index, edge_type, edge_norm,
           triples):
  n_ent, d_in = ent_emb.shape
  d_out = in_w.shape[1]
  n_rel = rel_emb.shape[0]
  n_edges = edge_norm.shape[0]
  batch = triples.shape[0]
  chunks = n_edges // _CH
  cpc = chunks // 2

  pad = 8

  def _chunked(a):
    a = a.reshape(chunks, _CH)
    return jnp.concatenate([a, jnp.zeros((pad, _CH), a.dtype)], axis=0)

  src_c = _chunked(edge_index[0])
  dst_c = _chunked(edge_index[1])
  typ_c = _chunked(edge_type)
  nrm_c = _chunked(edge_norm)
  zeros = jnp.zeros((n_ent, d_in), jnp.float32)

  edge_agg = _make_edge_agg(n_ent, d_in, n_rel, cpc)
  agg_in, agg_out = edge_agg(ent_emb, rel_emb, src_c, typ_c, dst_c, nrm_c,
                             zeros)

  nblk = 10
  rblk = n_ent // nblk
  xpre, stats, r = pl.pallas_call(
      functools.partial(_xpre_body, nblk=nblk),
      grid=(nblk,),
      in_specs=[
          pl.BlockSpec((rblk, d_in), lambda i: (i, 0)),
          pl.BlockSpec((rblk, d_in), lambda i: (i, 0)),
          pl.BlockSpec((rblk, d_in), lambda i: (i, 0)),
          pl.BlockSpec((d_in, d_out), lambda i: (0, 0)),
          pl.BlockSpec((d_in, d_out), lambda i: (0, 0)),
          pl.BlockSpec((d_in, d_out), lambda i: (0, 0)),
          pl.BlockSpec((1, d_in), lambda i: (0, 0)),
          pl.BlockSpec((1, d_out), lambda i: (0, 0)),
          pl.BlockSpec((n_rel, d_in), lambda i: (0, 0)),
          pl.BlockSpec((d_in, d_out), lambda i: (0, 0)),
      ],
      out_specs=[
          pl.BlockSpec((rblk, d_out), lambda i: (i, 0)),
          pl.BlockSpec((8, d_out), lambda i: (0, 0)),
          pl.BlockSpec((n_rel, d_out), lambda i: (0, 0)),
      ],
      out_shape=[
          jax.ShapeDtypeStruct((n_ent, d_out), jnp.float32),
          jax.ShapeDtypeStruct((8, d_out), jnp.float32),
          jax.ShapeDtypeStruct((n_rel, d_out), jnp.float32),
      ],
      scratch_shapes=[pltpu.VMEM((8, d_out), jnp.float32)],
  )(agg_in, agg_out, ent_emb, in_w, out_w, loop_w,
    loop_rel, bias_cov.reshape(1, d_out), rel_emb, w_rel)

  pair_gather = _make_pair_gather(d_out, batch)
  head = jnp.asarray(triples[:, 0], jnp.int32)
  rela = jnp.asarray(triples[:, 1], jnp.int32)
  xh, rh = pair_gather(xpre, r, head, rela)

  cblk = 1000
  ncb = n_ent // cblk
  score = pl.pallas_call(
      functools.partial(_decoder_body, n_ent=float(n_ent), cblk=cblk),
      grid=(ncb,),
      in_specs=[
          pl.BlockSpec((batch, d_out), lambda i: (0, 0)),
          pl.BlockSpec((batch, d_out), lambda i: (0, 0)),
          pl.BlockSpec((8, d_out), lambda i: (0, 0)),
          pl.BlockSpec((1, d_out), lambda i: (0, 0)),
          pl.BlockSpec((1, d_out), lambda i: (0, 0)),
          pl.BlockSpec((cblk, d_out), lambda i: (i, 0)),
          pl.BlockSpec((1, n_ent), lambda i: (0, 0)),
      ],
      out_specs=pl.BlockSpec((batch, cblk), lambda i: (0, i)),
      out_shape=jax.ShapeDtypeStruct((batch, n_ent), jnp.float32),
      scratch_shapes=[pltpu.VMEM((batch, d_out), jnp.float32)],
  )(xh, rh, stats, bn_gamma.reshape(1, d_out), bn_beta.reshape(1, d_out),
    emb_ent, b_ent.reshape(1, n_ent))

  return score


# speedup vs baseline: 4.8750x; 4.8750x over previous
import functools

import jax
import jax.numpy as jnp
from jax import lax
from jax.experimental import pallas as pl
from jax.experimental.pallas import tpu as pltpu
from jax.experimental.pallas import tpu_sc as plsc

_CH = 64


def _make_edge_agg(n_ent_pad, d_in, chunks_per_tile):
  mesh = plsc.VectorSubcoreMesh(core_axis_name="c", subcore_axis_name="s")
  ns = 16
  rows_per_tile = n_ent_pad // ns
  nlane = d_in // 16
  nchunks = chunks_per_tile

  @functools.partial(
      pl.kernel,
      mesh=mesh,
      out_type=[
          jax.ShapeDtypeStruct((n_ent_pad, d_in), jnp.float32),
          jax.ShapeDtypeStruct((n_ent_pad, d_in), jnp.float32),
      ],
      scratch_types=[
          pltpu.VMEM_SHARED((n_ent_pad, d_in), jnp.float32),
          pltpu.VMEM((2, _CH), jnp.int32),
          pltpu.VMEM((2, _CH), jnp.int32),
          pltpu.VMEM((2, _CH), jnp.int32),
          pltpu.VMEM((2, _CH), jnp.float32),
          pltpu.VMEM((2, _CH, d_in), jnp.float32),
          pltpu.VMEM((2, _CH, d_in), jnp.float32),
          pltpu.SemaphoreType.DMA,
          pltpu.SemaphoreType.DMA,
          pltpu.SemaphoreType.DMA,
      ],
  )
  def edge_agg(ent_hbm, rel_hbm, src_hbm, typ_hbm, dst_hbm, nrm_hbm, zero_hbm,
               out_in, out_out, shared, src_v, typ_v, dst_v, nrm_v,
               erow, rrow, sem_a, sem_b, sem_i):
    c = lax.axis_index("c")
    s = lax.axis_index("s")
    rb = s * rows_per_tile
    pltpu.sync_copy(zero_hbm.at[pl.ds(rb, rows_per_tile)],
                    shared.at[pl.ds(rb, rows_per_tile)])
    start = (c * ns + s) * nchunks
    plsc.subcore_barrier()

    def fire_idx(i, slot):
      base = (start + i) * _CH
      pltpu.async_copy(src_hbm.at[pl.ds(base, _CH)], src_v.at[slot], sem_i)
      pltpu.async_copy(typ_hbm.at[pl.ds(base, _CH)], typ_v.at[slot], sem_i)
      pltpu.async_copy(dst_hbm.at[pl.ds(base, _CH)], dst_v.at[slot], sem_i)
      pltpu.async_copy(nrm_hbm.at[pl.ds(base, _CH)], nrm_v.at[slot], sem_i)

    def wait_idx(slot):
      z = pl.ds(0, _CH)
      pltpu.make_async_copy(src_hbm.at[z], src_v.at[slot], sem_i).wait()
      pltpu.make_async_copy(typ_hbm.at[z], typ_v.at[slot], sem_i).wait()
      pltpu.make_async_copy(dst_hbm.at[z], dst_v.at[slot], sem_i).wait()
      pltpu.make_async_copy(nrm_hbm.at[z], nrm_v.at[slot], sem_i).wait()

    def fire_rows(slot):
      pltpu.async_copy(ent_hbm.at[src_v.at[slot]], erow.at[slot], sem_a)
      pltpu.async_copy(rel_hbm.at[typ_v.at[slot]], rrow.at[slot], sem_b)

    def drain_rows(slot):
      pltpu.make_async_copy(ent_hbm.at[src_v.at[0]], erow.at[slot], sem_a).wait()
      pltpu.make_async_copy(rel_hbm.at[typ_v.at[0]], rrow.at[slot], sem_b).wait()

    fire_idx(0, 0)
    wait_idx(0)
    fire_rows(0)
    fire_idx(1, 1)

    dnums = lax.GatherDimensionNumbers(
        offset_dims=(), collapsed_slice_dims=(0,), start_index_map=(0,))

    def chunk_body(i, carry):
      slot = lax.rem(i, 2)
      nxt = lax.rem(i + 1, 2)

      drain_rows(slot)

      @pl.when(i + 1 < nchunks)
      def _():
        wait_idx(nxt)
        fire_rows(nxt)

      def group_body(g, carry2):
        gv = nrm_v[slot, pl.ds(g * 16, 16)]
        for lane in range(16):
          lidx = jnp.full((16, 1), lane, jnp.int32)
          n = lax.gather(gv, lidx, dnums, slice_sizes=(1,),
                         mode=lax.GatherScatterMode.PROMISE_IN_BOUNDS)
          e = g * 16 + lane
          for j in range(nlane):
            sl = pl.ds(j * 16, 16)
            erow[slot, e, sl] = (erow[slot, e, sl] - rrow[slot, e, sl]) * n
        return carry2

      lax.fori_loop(0, _CH // 16, group_body, 0)
      pltpu.sync_copy(erow.at[slot], shared.at[dst_v.at[slot]], add=True)

      @pl.when(i + 2 < nchunks)
      def _():
        fire_idx(i + 2, slot)

      return carry

    lax.fori_loop(0, nchunks, chunk_body, 0)
    plsc.subcore_barrier()

    @pl.when(c == 0)
    def _():
      pltpu.sync_copy(shared.at[pl.ds(rb, rows_per_tile)],
                      out_in.at[pl.ds(rb, rows_per_tile)])

    @pl.when(c == 1)
    def _():
      pltpu.sync_copy(shared.at[pl.ds(rb, rows_per_tile)],
                      out_out.at[pl.ds(rb, rows_per_tile)])

  return edge_agg


def _make_pair_gather(d, batch):
  mesh = plsc.VectorSubcoreMesh(core_axis_name="c", subcore_axis_name="s")
  nw = 32
  per = batch // nw

  @functools.partial(
      pl.kernel,
      mesh=mesh,
      out_type=[
          jax.ShapeDtypeStruct((batch, d), jnp.float32),
          jax.ShapeDtypeStruct((batch, d), jnp.float32),
      ],
      scratch_types=[
          pltpu.VMEM((per,), jnp.int32),
          pltpu.VMEM((per,), jnp.int32),
          pltpu.VMEM((per, d), jnp.float32),
          pltpu.VMEM((per, d), jnp.float32),
          pltpu.SemaphoreType.DMA,
          pltpu.SemaphoreType.DMA,
      ],
  )
  def pair_gather(x_hbm, r_hbm, head_hbm, rela_hbm, out_x, out_r,
                  hidx, ridx, xrow, rrow, sem_a, sem_b):
    c = lax.axis_index("c")
    s = lax.axis_index("s")
    base = (s * 2 + c) * per
    pltpu.sync_copy(head_hbm.at[pl.ds(base, per)], hidx)
    pltpu.sync_copy(rela_hbm.at[pl.ds(base, per)], ridx)
    ga = pltpu.async_copy(x_hbm.at[hidx], xrow, sem_a)
    gb = pltpu.async_copy(r_hbm.at[ridx], rrow, sem_b)
    ga.wait()
    gb.wait()
    pltpu.sync_copy(xrow, out_x.at[pl.ds(base, per)])
    pltpu.sync_copy(rrow, out_r.at[pl.ds(base, per)])

  return pair_gather


def _xpre_body(agg_in_ref, agg_out_ref, ent_ref, in_w_ref, out_w_ref,
               loop_w_ref, loop_rel_ref, bias_ref, rel_ref, w_rel_ref,
               xpre_ref, stats_ref, r_ref, acc, *, nblk):
  i = pl.program_id(0)
  f32 = jnp.float32
  xp = jnp.dot(agg_in_ref[...], in_w_ref[...], preferred_element_type=f32)
  xp += jnp.dot(agg_out_ref[...], out_w_ref[...], preferred_element_type=f32)
  xp += jnp.dot(ent_ref[...] - loop_rel_ref[...], loop_w_ref[...],
                preferred_element_type=f32)
  xp = xp * (1.0 / 3.0) + bias_ref[...]
  xpre_ref[...] = xp

  @pl.when(i == 0)
  def _():
    acc[...] = jnp.zeros_like(acc)
    r_ref[...] = jnp.dot(rel_ref[...], w_rel_ref[...], preferred_element_type=f32)

  acc[0:1, :] += jnp.sum(xp, axis=0, keepdims=True)
  acc[1:2, :] += jnp.sum(xp * xp, axis=0, keepdims=True)

  @pl.when(i == nblk - 1)
  def _():
    stats_ref[...] = acc[...]


def _decoder_body(xh_ref, rh_ref, stats_ref, gamma_ref, beta_ref,
                  emb_ref, bent_ref, out_ref, obj, *, n_ent, cblk):
  i = pl.program_id(0)

  @pl.when(i == 0)
  def _():
    inv_n = 1.0 / n_ent
    mean = stats_ref[0:1, :] * inv_n
    var = stats_ref[1:2, :] * inv_n - mean * mean
    xn = (xh_ref[...] - mean) * lax.rsqrt(var + 1e-5)
    xn = jnp.tanh(xn * gamma_ref[...] + beta_ref[...])
    obj[...] = xn * rh_ref[...]

  logits = lax.dot_general(obj[...], emb_ref[...], (((1,), (1,)), ((), ())),
                           preferred_element_type=jnp.float32)
  logits += bent_ref[0:1, pl.ds(i * cblk, cblk)]
  out_ref[...] = jax.nn.sigmoid(logits)


def kernel(ent_emb, rel_emb, in_w, out_w, loop_w, w_rel, loop_rel, bias_cov,
           bn_gamma, bn_beta, b_ent, emb_ent, edge_index, edge_type, edge_norm,
           triples):
  n_ent, d_in = ent_emb.shape
  d_out = in_w.shape[1]
  n_rel = rel_emb.shape[0]
  n_edges = edge_norm.shape[0]
  batch = triples.shape[0]
  chunks = n_edges // _CH
  cpc = chunks // 2
  ns = 16
  cpt = -(-cpc // ns)
  cpt = -(-cpt // 8) * 8
  cpc_pad = cpt * ns

  half = n_edges // 2
  npad = (cpc_pad - cpc) * _CH

  def _chunked(a, fill):
    zpad = jnp.full((npad,), fill, a.dtype)
    return jnp.concatenate([a[:half], zpad, a[half:], zpad])

  src_c = _chunked(edge_index[0], 0)
  dst_c = _chunked(edge_index[1], 0)
  typ_c = _chunked(edge_type, 0)
  nrm_c = _chunked(edge_norm, 0.0)
  n_ent_pad = -(-n_ent // (8 * ns)) * (8 * ns)
  zeros = jnp.zeros((n_ent_pad, d_in), jnp.float32)

  edge_agg = _make_edge_agg(n_ent_pad, d_in, cpt)
  agg_in, agg_out = edge_agg(ent_emb, rel_emb, src_c, typ_c, dst_c, nrm_c,
                             zeros)
  agg_in = agg_in[:n_ent]
  agg_out = agg_out[:n_ent]

  nblk = 10
  rblk = n_ent // nblk
  xpre, stats, r = pl.pallas_call(
      functools.partial(_xpre_body, nblk=nblk),
      grid=(nblk,),
      in_specs=[
          pl.BlockSpec((rblk, d_in), lambda i: (i, 0)),
          pl.BlockSpec((rblk, d_in), lambda i: (i, 0)),
          pl.BlockSpec((rblk, d_in), lambda i: (i, 0)),
          pl.BlockSpec((d_in, d_out), lambda i: (0, 0)),
          pl.BlockSpec((d_in, d_out), lambda i: (0, 0)),
          pl.BlockSpec((d_in, d_out), lambda i: (0, 0)),
          pl.BlockSpec((1, d_in), lambda i: (0, 0)),
          pl.BlockSpec((1, d_out), lambda i: (0, 0)),
          pl.BlockSpec((n_rel, d_in), lambda i: (0, 0)),
          pl.BlockSpec((d_in, d_out), lambda i: (0, 0)),
      ],
      out_specs=[
          pl.BlockSpec((rblk, d_out), lambda i: (i, 0)),
          pl.BlockSpec((8, d_out), lambda i: (0, 0)),
          pl.BlockSpec((n_rel, d_out), lambda i: (0, 0)),
      ],
      out_shape=[
          jax.ShapeDtypeStruct((n_ent, d_out), jnp.float32),
          jax.ShapeDtypeStruct((8, d_out), jnp.float32),
          jax.ShapeDtypeStruct((n_rel, d_out), jnp.float32),
      ],
      scratch_shapes=[pltpu.VMEM((8, d_out), jnp.float32)],
  )(agg_in, agg_out, ent_emb, in_w, out_w, loop_w,
    loop_rel, bias_cov.reshape(1, d_out), rel_emb, w_rel)

  pair_gather = _make_pair_gather(d_out, batch)
  head = jnp.asarray(triples[:, 0], jnp.int32)
  rela = jnp.asarray(triples[:, 1], jnp.int32)
  xh, rh = pair_gather(xpre, r, head, rela)

  cblk = 1024
  n_pad = 10240 if n_ent == 10000 else ((n_ent + cblk - 1) // cblk) * cblk
  ncb = n_pad // cblk
  emb_p = jnp.concatenate(
      [emb_ent, jnp.zeros((n_pad - n_ent, d_out), jnp.float32)], axis=0)
  bent_p = jnp.concatenate(
      [b_ent, jnp.zeros((n_pad - n_ent,), jnp.float32)]).reshape(1, n_pad)
  score = pl.pallas_call(
      functools.partial(_decoder_body, n_ent=float(n_ent), cblk=cblk),
      grid=(ncb,),
      in_specs=[
          pl.BlockSpec((batch, d_out), lambda i: (0, 0)),
          pl.BlockSpec((batch, d_out), lambda i: (0, 0)),
          pl.BlockSpec((8, d_out), lambda i: (0, 0)),
          pl.BlockSpec((1, d_out), lambda i: (0, 0)),
          pl.BlockSpec((1, d_out), lambda i: (0, 0)),
          pl.BlockSpec((cblk, d_out), lambda i: (i, 0)),
          pl.BlockSpec((1, n_pad), lambda i: (0, 0)),
      ],
      out_specs=pl.BlockSpec((batch, cblk), lambda i: (0, i)),
      out_shape=jax.ShapeDtypeStruct((batch, n_pad), jnp.float32),
      scratch_shapes=[pltpu.VMEM((batch, d_out), jnp.float32)],
  )(xh, rh, stats, bn_gamma.reshape(1, d_out), bn_beta.reshape(1, d_out),
    emb_p, bent_p)

  return score[:, :n_ent]


# speedup vs baseline: 5.1313x; 1.0526x over previous
import functools

import jax
import jax.numpy as jnp
from jax import lax
from jax.experimental import pallas as pl
from jax.experimental.pallas import tpu as pltpu
from jax.experimental.pallas import tpu_sc as plsc

_CH = 64


def _make_edge_agg(n_ent_pad, d_in, chunks_per_tile):
  mesh = plsc.VectorSubcoreMesh(core_axis_name="c", subcore_axis_name="s")
  ns = 16
  rows_per_tile = n_ent_pad // ns
  nlane = d_in // 16
  nchunks = chunks_per_tile

  @functools.partial(
      pl.kernel,
      mesh=mesh,
      out_type=[
          jax.ShapeDtypeStruct((n_ent_pad, d_in), jnp.float32),
          jax.ShapeDtypeStruct((n_ent_pad, d_in), jnp.float32),
      ],
      scratch_types=[
          pltpu.VMEM_SHARED((n_ent_pad, d_in), jnp.float32),
          pltpu.VMEM((2, _CH), jnp.int32),
          pltpu.VMEM((2, _CH), jnp.int32),
          pltpu.VMEM((3, _CH), jnp.int32),
          pltpu.VMEM((2, _CH), jnp.float32),
          pltpu.VMEM((2, _CH, d_in), jnp.float32),
          pltpu.VMEM((2, _CH, d_in), jnp.float32),
          pltpu.SemaphoreType.DMA,
          pltpu.SemaphoreType.DMA,
          pltpu.SemaphoreType.DMA,
          pltpu.SemaphoreType.DMA,
      ],
  )
  def edge_agg(ent_hbm, rel_hbm, src_hbm, typ_hbm, dst_hbm, nrm_hbm, zero_hbm,
               out_in, out_out, shared, src_v, typ_v, dst_v, nrm_v,
               erow, rrow, sem_a, sem_b, sem_i, sem_s):
    c = lax.axis_index("c")
    s = lax.axis_index("s")
    rb = s * rows_per_tile
    pltpu.sync_copy(zero_hbm.at[pl.ds(rb, rows_per_tile)],
                    shared.at[pl.ds(rb, rows_per_tile)])
    start = (c * ns + s) * nchunks
    plsc.subcore_barrier()

    def fire_idx(i, slot, dslot):
      base = (start + i) * _CH
      pltpu.async_copy(src_hbm.at[pl.ds(base, _CH)], src_v.at[slot], sem_i)
      pltpu.async_copy(typ_hbm.at[pl.ds(base, _CH)], typ_v.at[slot], sem_i)
      pltpu.async_copy(dst_hbm.at[pl.ds(base, _CH)], dst_v.at[dslot], sem_i)
      pltpu.async_copy(nrm_hbm.at[pl.ds(base, _CH)], nrm_v.at[slot], sem_i)

    def wait_idx(slot, dslot):
      z = pl.ds(0, _CH)
      pltpu.make_async_copy(src_hbm.at[z], src_v.at[slot], sem_i).wait()
      pltpu.make_async_copy(typ_hbm.at[z], typ_v.at[slot], sem_i).wait()
      pltpu.make_async_copy(dst_hbm.at[z], dst_v.at[dslot], sem_i).wait()
      pltpu.make_async_copy(nrm_hbm.at[z], nrm_v.at[slot], sem_i).wait()

    def wait_scatter(dslot):
      pltpu.make_async_copy(erow.at[0], shared.at[dst_v.at[dslot]],
                            sem_s).wait()

    def fire_rows(slot):
      pltpu.async_copy(ent_hbm.at[src_v.at[slot]], erow.at[slot], sem_a)
      pltpu.async_copy(rel_hbm.at[typ_v.at[slot]], rrow.at[slot], sem_b)

    def drain_rows(slot):
      pltpu.make_async_copy(ent_hbm.at[src_v.at[0]], erow.at[slot], sem_a).wait()
      pltpu.make_async_copy(rel_hbm.at[typ_v.at[0]], rrow.at[slot], sem_b).wait()

    fire_idx(0, 0, 0)
    wait_idx(0, 0)
    fire_rows(0)
    fire_idx(1, 1, 1)

    dnums = lax.GatherDimensionNumbers(
        offset_dims=(), collapsed_slice_dims=(0,), start_index_map=(0,))

    def chunk_body(i, carry):
      slot = lax.rem(i, 2)
      nxt = lax.rem(i + 1, 2)
      dslot = lax.rem(i, 3)

      drain_rows(slot)

      @pl.when(i >= 1)
      def _():
        wait_scatter(lax.rem(i + 2, 3))

      @pl.when(i + 1 < nchunks)
      def _():
        wait_idx(nxt, lax.rem(i + 1, 3))
        fire_rows(nxt)

      def group_body(g, carry2):
        gv = nrm_v[slot, pl.ds(g * 16, 16)]
        for lane in range(16):
          lidx = jnp.full((16, 1), lane, jnp.int32)
          n = lax.gather(gv, lidx, dnums, slice_sizes=(1,),
                         mode=lax.GatherScatterMode.PROMISE_IN_BOUNDS)
          e = g * 16 + lane
          for j in range(nlane):
            sl = pl.ds(j * 16, 16)
            erow[slot, e, sl] = (erow[slot, e, sl] - rrow[slot, e, sl]) * n
        return carry2

      lax.fori_loop(0, _CH // 16, group_body, 0)
      pltpu.async_copy(erow.at[slot], shared.at[dst_v.at[dslot]], sem_s,
                       add=True)

      @pl.when(i + 2 < nchunks)
      def _():
        fire_idx(i + 2, slot, lax.rem(i + 2, 3))

      return carry

    lax.fori_loop(0, nchunks, chunk_body, 0)
    wait_scatter(lax.rem(nchunks - 1, 3))
    plsc.subcore_barrier()

    @pl.when(c == 0)
    def _():
      pltpu.sync_copy(shared.at[pl.ds(rb, rows_per_tile)],
                      out_in.at[pl.ds(rb, rows_per_tile)])

    @pl.when(c == 1)
    def _():
      pltpu.sync_copy(shared.at[pl.ds(rb, rows_per_tile)],
                      out_out.at[pl.ds(rb, rows_per_tile)])

  return edge_agg


def _make_pair_gather(d, batch):
  mesh = plsc.VectorSubcoreMesh(core_axis_name="c", subcore_axis_name="s")
  nw = 32
  per = batch // nw

  @functools.partial(
      pl.kernel,
      mesh=mesh,
      out_type=[
          jax.ShapeDtypeStruct((batch, d), jnp.float32),
          jax.ShapeDtypeStruct((batch, d), jnp.float32),
      ],
      scratch_types=[
          pltpu.VMEM((per,), jnp.int32),
          pltpu.VMEM((per,), jnp.int32),
          pltpu.VMEM((per, d), jnp.float32),
          pltpu.VMEM((per, d), jnp.float32),
          pltpu.SemaphoreType.DMA,
          pltpu.SemaphoreType.DMA,
      ],
  )
  def pair_gather(x_hbm, r_hbm, head_hbm, rela_hbm, out_x, out_r,
                  hidx, ridx, xrow, rrow, sem_a, sem_b):
    c = lax.axis_index("c")
    s = lax.axis_index("s")
    base = (s * 2 + c) * per
    pltpu.sync_copy(head_hbm.at[pl.ds(base, per)], hidx)
    pltpu.sync_copy(rela_hbm.at[pl.ds(base, per)], ridx)
    ga = pltpu.async_copy(x_hbm.at[hidx], xrow, sem_a)
    gb = pltpu.async_copy(r_hbm.at[ridx], rrow, sem_b)
    ga.wait()
    gb.wait()
    pltpu.sync_copy(xrow, out_x.at[pl.ds(base, per)])
    pltpu.sync_copy(rrow, out_r.at[pl.ds(base, per)])

  return pair_gather


def _xpre_body(agg_in_ref, agg_out_ref, ent_ref, in_w_ref, out_w_ref,
               loop_w_ref, loop_rel_ref, bias_ref, rel_ref, w_rel_ref,
               xpre_ref, stats_ref, r_ref, acc, *, nblk):
  i = pl.program_id(0)
  f32 = jnp.float32
  xp = jnp.dot(agg_in_ref[...], in_w_ref[...], preferred_element_type=f32)
  xp += jnp.dot(agg_out_ref[...], out_w_ref[...], preferred_element_type=f32)
  xp += jnp.dot(ent_ref[...] - loop_rel_ref[...], loop_w_ref[...],
                preferred_element_type=f32)
  xp = xp * (1.0 / 3.0) + bias_ref[...]
  xpre_ref[...] = xp

  @pl.when(i == 0)
  def _():
    acc[...] = jnp.zeros_like(acc)
    r_ref[...] = jnp.dot(rel_ref[...], w_rel_ref[...], preferred_element_type=f32)

  acc[0:1, :] += jnp.sum(xp, axis=0, keepdims=True)
  acc[1:2, :] += jnp.sum(xp * xp, axis=0, keepdims=True)

  @pl.when(i == nblk - 1)
  def _():
    stats_ref[...] = acc[...]


def _decoder_body(xh_ref, rh_ref, stats_ref, gamma_ref, beta_ref,
                  emb_ref, bent_ref, out_ref, obj, *, n_ent, cblk):
  i = pl.program_id(0)

  @pl.when(i == 0)
  def _():
    inv_n = 1.0 / n_ent
    mean = stats_ref[0:1, :] * inv_n
    var = stats_ref[1:2, :] * inv_n - mean * mean
    xn = (xh_ref[...] - mean) * lax.rsqrt(var + 1e-5)
    xn = jnp.tanh(xn * gamma_ref[...] + beta_ref[...])
    obj[...] = xn * rh_ref[...]

  logits = lax.dot_general(obj[...], emb_ref[...], (((1,), (1,)), ((), ())),
                           preferred_element_type=jnp.float32)
  logits += bent_ref[0:1, pl.ds(i * cblk, cblk)]
  out_ref[...] = jax.nn.sigmoid(logits)


def kernel(ent_emb, rel_emb, in_w, out_w, loop_w, w_rel, loop_rel, bias_cov,
           bn_gamma, bn_beta, b_ent, emb_ent, edge_index, edge_type, edge_norm,
           triples):
  n_ent, d_in = ent_emb.shape
  d_out = in_w.shape[1]
  n_rel = rel_emb.shape[0]
  n_edges = edge_norm.shape[0]
  batch = triples.shape[0]
  chunks = n_edges // _CH
  cpc = chunks // 2
  ns = 16
  cpt = -(-cpc // ns)
  cpt = -(-cpt // 8) * 8
  cpc_pad = cpt * ns

  half = n_edges // 2
  npad = (cpc_pad - cpc) * _CH

  def _chunked(a, fill):
    zpad = jnp.full((npad,), fill, a.dtype)
    return jnp.concatenate([a[:half], zpad, a[half:], zpad])

  src_c = _chunked(edge_index[0], 0)
  dst_c = _chunked(edge_index[1], 0)
  typ_c = _chunked(edge_type, 0)
  nrm_c = _chunked(edge_norm, 0.0)
  n_ent_pad = -(-n_ent // (8 * ns)) * (8 * ns)
  zeros = jnp.zeros((n_ent_pad, d_in), jnp.float32)

  edge_agg = _make_edge_agg(n_ent_pad, d_in, cpt)
  agg_in, agg_out = edge_agg(ent_emb, rel_emb, src_c, typ_c, dst_c, nrm_c,
                             zeros)
  agg_in = agg_in[:n_ent]
  agg_out = agg_out[:n_ent]

  nblk = 10
  rblk = n_ent // nblk
  xpre, stats, r = pl.pallas_call(
      functools.partial(_xpre_body, nblk=nblk),
      grid=(nblk,),
      in_specs=[
          pl.BlockSpec((rblk, d_in), lambda i: (i, 0)),
          pl.BlockSpec((rblk, d_in), lambda i: (i, 0)),
          pl.BlockSpec((rblk, d_in), lambda i: (i, 0)),
          pl.BlockSpec((d_in, d_out), lambda i: (0, 0)),
          pl.BlockSpec((d_in, d_out), lambda i: (0, 0)),
          pl.BlockSpec((d_in, d_out), lambda i: (0, 0)),
          pl.BlockSpec((1, d_in), lambda i: (0, 0)),
          pl.BlockSpec((1, d_out), lambda i: (0, 0)),
          pl.BlockSpec((n_rel, d_in), lambda i: (0, 0)),
          pl.BlockSpec((d_in, d_out), lambda i: (0, 0)),
      ],
      out_specs=[
          pl.BlockSpec((rblk, d_out), lambda i: (i, 0)),
          pl.BlockSpec((8, d_out), lambda i: (0, 0)),
          pl.BlockSpec((n_rel, d_out), lambda i: (0, 0)),
      ],
      out_shape=[
          jax.ShapeDtypeStruct((n_ent, d_out), jnp.float32),
          jax.ShapeDtypeStruct((8, d_out), jnp.float32),
          jax.ShapeDtypeStruct((n_rel, d_out), jnp.float32),
      ],
      scratch_shapes=[pltpu.VMEM((8, d_out), jnp.float32)],
  )(agg_in, agg_out, ent_emb, in_w, out_w, loop_w,
    loop_rel, bias_cov.reshape(1, d_out), rel_emb, w_rel)

  pair_gather = _make_pair_gather(d_out, batch)
  head = jnp.asarray(triples[:, 0], jnp.int32)
  rela = jnp.asarray(triples[:, 1], jnp.int32)
  xh, rh = pair_gather(xpre, r, head, rela)

  cblk = 1024
  n_pad = 10240 if n_ent == 10000 else ((n_ent + cblk - 1) // cblk) * cblk
  ncb = n_pad // cblk
  emb_p = jnp.concatenate(
      [emb_ent, jnp.zeros((n_pad - n_ent, d_out), jnp.float32)], axis=0)
  bent_p = jnp.concatenate(
      [b_ent, jnp.zeros((n_pad - n_ent,), jnp.float32)]).reshape(1, n_pad)
  score = pl.pallas_call(
      functools.partial(_decoder_body, n_ent=float(n_ent), cblk=cblk),
      grid=(ncb,),
      in_specs=[
          pl.BlockSpec((batch, d_out), lambda i: (0, 0)),
          pl.BlockSpec((batch, d_out), lambda i: (0, 0)),
          pl.BlockSpec((8, d_out), lambda i: (0, 0)),
          pl.BlockSpec((1, d_out), lambda i: (0, 0)),
          pl.BlockSpec((1, d_out), lambda i: (0, 0)),
          pl.BlockSpec((cblk, d_out), lambda i: (i, 0)),
          pl.BlockSpec((1, n_pad), lambda i: (0, 0)),
      ],
      out_specs=pl.BlockSpec((batch, cblk), lambda i: (0, i)),
      out_shape=jax.ShapeDtypeStruct((batch, n_pad), jnp.float32),
      scratch_shapes=[pltpu.VMEM((batch, d_out), jnp.float32)],
  )(xh, rh, stats, bn_gamma.reshape(1, d_out), bn_beta.reshape(1, d_out),
    emb_p, bent_p)

  return score[:, :n_ent]


# speedup vs baseline: 5.3876x; 1.0500x over previous
import functools

import jax
import jax.numpy as jnp
from jax import lax
from jax.experimental import pallas as pl
from jax.experimental.pallas import tpu as pltpu
from jax.experimental.pallas import tpu_sc as plsc

_CH = 64


def _make_edge_agg(n_ent_pad, d_in, chunks_per_tile):
  mesh = plsc.VectorSubcoreMesh(core_axis_name="c", subcore_axis_name="s")
  ns = 16
  rows_per_tile = n_ent_pad // ns
  nlane = d_in // 16
  nchunks = chunks_per_tile

  @functools.partial(
      pl.kernel,
      mesh=mesh,
      out_type=[
          jax.ShapeDtypeStruct((n_ent_pad, d_in), jnp.float32),
          jax.ShapeDtypeStruct((n_ent_pad, d_in), jnp.float32),
      ],
      scratch_types=[
          pltpu.VMEM_SHARED((n_ent_pad, d_in), jnp.float32),
          pltpu.VMEM((2, _CH), jnp.int32),
          pltpu.VMEM((2, _CH), jnp.int32),
          pltpu.VMEM((3, _CH), jnp.int32),
          pltpu.VMEM((2, _CH), jnp.float32),
          pltpu.VMEM((2, _CH, d_in), jnp.float32),
          pltpu.VMEM((2, _CH, d_in), jnp.float32),
          pltpu.SemaphoreType.DMA,
          pltpu.SemaphoreType.DMA,
          pltpu.SemaphoreType.DMA,
          pltpu.SemaphoreType.DMA,
      ],
  )
  def edge_agg(ent_hbm, rel_hbm, src_hbm, typ_hbm, dst_hbm, nrm_hbm, zero_hbm,
               out_in, out_out, shared, src_v, typ_v, dst_v, nrm_v,
               erow, rrow, sem_a, sem_b, sem_i, sem_s):
    c = lax.axis_index("c")
    s = lax.axis_index("s")
    rb = s * rows_per_tile
    pltpu.sync_copy(zero_hbm.at[pl.ds(rb, rows_per_tile)],
                    shared.at[pl.ds(rb, rows_per_tile)])
    start = (c * ns + s) * nchunks
    plsc.subcore_barrier()

    def fire_idx(i, slot, dslot):
      base = (start + i) * _CH
      pltpu.async_copy(src_hbm.at[pl.ds(base, _CH)], src_v.at[slot], sem_i)
      pltpu.async_copy(typ_hbm.at[pl.ds(base, _CH)], typ_v.at[slot], sem_i)
      pltpu.async_copy(dst_hbm.at[pl.ds(base, _CH)], dst_v.at[dslot], sem_i)
      pltpu.async_copy(nrm_hbm.at[pl.ds(base, _CH)], nrm_v.at[slot], sem_i)

    def wait_idx(slot, dslot):
      z = pl.ds(0, _CH)
      pltpu.make_async_copy(src_hbm.at[z], src_v.at[slot], sem_i).wait()
      pltpu.make_async_copy(typ_hbm.at[z], typ_v.at[slot], sem_i).wait()
      pltpu.make_async_copy(dst_hbm.at[z], dst_v.at[dslot], sem_i).wait()
      pltpu.make_async_copy(nrm_hbm.at[z], nrm_v.at[slot], sem_i).wait()

    def wait_scatter(dslot):
      pltpu.make_async_copy(erow.at[0], shared.at[dst_v.at[dslot]],
                            sem_s).wait()

    def fire_rows(slot):
      pltpu.async_copy(ent_hbm.at[src_v.at[slot]], erow.at[slot], sem_a)
      pltpu.async_copy(rel_hbm.at[typ_v.at[slot]], rrow.at[slot], sem_b)

    def drain_rows(slot):
      pltpu.make_async_copy(ent_hbm.at[src_v.at[0]], erow.at[slot], sem_a).wait()
      pltpu.make_async_copy(rel_hbm.at[typ_v.at[0]], rrow.at[slot], sem_b).wait()

    fire_idx(0, 0, 0)
    wait_idx(0, 0)
    fire_rows(0)
    fire_idx(1, 1, 1)

    dnums = lax.GatherDimensionNumbers(
        offset_dims=(), collapsed_slice_dims=(0,), start_index_map=(0,))

    def chunk_body(i, carry):
      slot = lax.rem(i, 2)
      nxt = lax.rem(i + 1, 2)
      dslot = lax.rem(i, 3)

      drain_rows(slot)

      @pl.when(i >= 1)
      def _():
        wait_scatter(lax.rem(i + 2, 3))

      @pl.when(i + 1 < nchunks)
      def _():
        wait_idx(nxt, lax.rem(i + 1, 3))
        fire_rows(nxt)

      def group_body(g, carry2):
        gv = nrm_v[slot, pl.ds(g * 16, 16)]
        for lane in range(16):
          lidx = jnp.full((16, 1), lane, jnp.int32)
          n = lax.gather(gv, lidx, dnums, slice_sizes=(1,),
                         mode=lax.GatherScatterMode.PROMISE_IN_BOUNDS)
          e = g * 16 + lane
          for j in range(nlane):
            sl = pl.ds(j * 16, 16)
            erow[slot, e, sl] = (erow[slot, e, sl] - rrow[slot, e, sl]) * n
        return carry2

      lax.fori_loop(0, _CH // 16, group_body, 0)
      pltpu.async_copy(erow.at[slot], shared.at[dst_v.at[dslot]], sem_s,
                       add=True)

      @pl.when(i + 2 < nchunks)
      def _():
        fire_idx(i + 2, slot, lax.rem(i + 2, 3))

      return carry

    lax.fori_loop(0, nchunks, chunk_body, 0)
    wait_scatter(lax.rem(nchunks - 1, 3))
    plsc.subcore_barrier()

    @pl.when(c == 0)
    def _():
      pltpu.sync_copy(shared.at[pl.ds(rb, rows_per_tile)],
                      out_in.at[pl.ds(rb, rows_per_tile)])

    @pl.when(c == 1)
    def _():
      pltpu.sync_copy(shared.at[pl.ds(rb, rows_per_tile)],
                      out_out.at[pl.ds(rb, rows_per_tile)])

  return edge_agg


def _make_pair_gather(d, batch):
  mesh = plsc.VectorSubcoreMesh(core_axis_name="c", subcore_axis_name="s")
  nw = 32
  per = batch // nw

  @functools.partial(
      pl.kernel,
      mesh=mesh,
      out_type=[
          jax.ShapeDtypeStruct((batch, d), jnp.float32),
          jax.ShapeDtypeStruct((batch, d), jnp.float32),
      ],
      scratch_types=[
          pltpu.VMEM((per,), jnp.int32),
          pltpu.VMEM((per,), jnp.int32),
          pltpu.VMEM((per, d), jnp.float32),
          pltpu.VMEM((per, d), jnp.float32),
          pltpu.SemaphoreType.DMA,
          pltpu.SemaphoreType.DMA,
      ],
  )
  def pair_gather(x_hbm, r_hbm, head_hbm, rela_hbm, out_x, out_r,
                  hidx, ridx, xrow, rrow, sem_a, sem_b):
    c = lax.axis_index("c")
    s = lax.axis_index("s")
    base = (s * 2 + c) * per
    pltpu.sync_copy(head_hbm.at[pl.ds(base, per)], hidx)
    pltpu.sync_copy(rela_hbm.at[pl.ds(base, per)], ridx)
    ga = pltpu.async_copy(x_hbm.at[hidx], xrow, sem_a)
    gb = pltpu.async_copy(r_hbm.at[ridx], rrow, sem_b)
    ga.wait()
    gb.wait()
    pltpu.sync_copy(xrow, out_x.at[pl.ds(base, per)])
    pltpu.sync_copy(rrow, out_r.at[pl.ds(base, per)])

  return pair_gather


def _xpre_body(agg_in_ref, agg_out_ref, ent_ref, in_w_ref, out_w_ref,
               loop_w_ref, loop_rel_ref, bias_ref, rel_ref, w_rel_ref,
               xpre_ref, stats_ref, r_ref, acc, *, nblk, rblk, n_ent):
  i = pl.program_id(0)
  f32 = jnp.float32
  xp = jnp.dot(agg_in_ref[...], in_w_ref[...], preferred_element_type=f32)
  xp += jnp.dot(agg_out_ref[...], out_w_ref[...], preferred_element_type=f32)
  xp += jnp.dot(ent_ref[...] - loop_rel_ref[...], loop_w_ref[...],
                preferred_element_type=f32)
  xp = xp * (1.0 / 3.0) + bias_ref[...]
  xpre_ref[...] = xp

  @pl.when(i == 0)
  def _():
    acc[...] = jnp.zeros_like(acc)
    r_ref[...] = jnp.dot(rel_ref[...], w_rel_ref[...], preferred_element_type=f32)

  row = i * rblk + lax.broadcasted_iota(jnp.int32, xp.shape, 0)
  xpm = jnp.where(row < n_ent, xp, 0.0)
  acc[0:1, :] += jnp.sum(xpm, axis=0, keepdims=True)
  acc[1:2, :] += jnp.sum(xpm * xpm, axis=0, keepdims=True)

  @pl.when(i == nblk - 1)
  def _():
    stats_ref[...] = acc[...]


def _decoder_body(xh_ref, rh_ref, stats_ref, gamma_ref, beta_ref,
                  emb_ref, bent_ref, out_ref, obj, *, n_ent, bblk):
  i = pl.program_id(0)

  @pl.when(i == 0)
  def _():
    inv_n = 1.0 / n_ent
    mean = stats_ref[0:1, :] * inv_n
    var = stats_ref[1:2, :] * inv_n - mean * mean
    xn = (xh_ref[...] - mean) * lax.rsqrt(var + 1e-5)
    xn = jnp.tanh(xn * gamma_ref[...] + beta_ref[...])
    obj[...] = xn * rh_ref[...]

  logits = lax.dot_general(obj[pl.ds(i * bblk, bblk), :], emb_ref[...],
                           (((1,), (1,)), ((), ())),
                           preferred_element_type=jnp.float32)
  logits += bent_ref[...]
  out_ref[...] = jax.nn.sigmoid(logits)


def kernel(ent_emb, rel_emb, in_w, out_w, loop_w, w_rel, loop_rel, bias_cov,
           bn_gamma, bn_beta, b_ent, emb_ent, edge_index, edge_type, edge_norm,
           triples):
  n_ent, d_in = ent_emb.shape
  d_out = in_w.shape[1]
  n_rel = rel_emb.shape[0]
  n_edges = edge_norm.shape[0]
  batch = triples.shape[0]
  chunks = n_edges // _CH
  cpc = chunks // 2
  ns = 16
  cpt = -(-cpc // ns)
  cpt = -(-cpt // 8) * 8
  cpc_pad = cpt * ns

  half = n_edges // 2
  npad = (cpc_pad - cpc) * _CH

  def _chunked(a, fill):
    zpad = jnp.full((npad,), fill, a.dtype)
    return jnp.concatenate([a[:half], zpad, a[half:], zpad])

  src_c = _chunked(edge_index[0], 0)
  dst_c = _chunked(edge_index[1], 0)
  typ_c = _chunked(edge_type, 0)
  nrm_c = _chunked(edge_norm, 0.0)
  n_ent_pad = -(-n_ent // (80 * ns)) * (80 * ns)
  zeros = jnp.zeros((n_ent_pad, d_in), jnp.float32)

  edge_agg = _make_edge_agg(n_ent_pad, d_in, cpt)
  agg_in, agg_out = edge_agg(ent_emb, rel_emb, src_c, typ_c, dst_c, nrm_c,
                             zeros)

  ent_p = jnp.concatenate(
      [ent_emb, jnp.zeros((n_ent_pad - n_ent, d_in), jnp.float32)], axis=0)
  rblk = 1024
  nblk = n_ent_pad // rblk
  xpre, stats, r = pl.pallas_call(
      functools.partial(_xpre_body, nblk=nblk, rblk=rblk, n_ent=n_ent),
      grid=(nblk,),
      in_specs=[
          pl.BlockSpec((rblk, d_in), lambda i: (i, 0)),
          pl.BlockSpec((rblk, d_in), lambda i: (i, 0)),
          pl.BlockSpec((rblk, d_in), lambda i: (i, 0)),
          pl.BlockSpec((d_in, d_out), lambda i: (0, 0)),
          pl.BlockSpec((d_in, d_out), lambda i: (0, 0)),
          pl.BlockSpec((d_in, d_out), lambda i: (0, 0)),
          pl.BlockSpec((1, d_in), lambda i: (0, 0)),
          pl.BlockSpec((1, d_out), lambda i: (0, 0)),
          pl.BlockSpec((n_rel, d_in), lambda i: (0, 0)),
          pl.BlockSpec((d_in, d_out), lambda i: (0, 0)),
      ],
      out_specs=[
          pl.BlockSpec((rblk, d_out), lambda i: (i, 0)),
          pl.BlockSpec((8, d_out), lambda i: (0, 0)),
          pl.BlockSpec((n_rel, d_out), lambda i: (0, 0)),
      ],
      out_shape=[
          jax.ShapeDtypeStruct((n_ent_pad, d_out), jnp.float32),
          jax.ShapeDtypeStruct((8, d_out), jnp.float32),
          jax.ShapeDtypeStruct((n_rel, d_out), jnp.float32),
      ],
      scratch_shapes=[pltpu.VMEM((8, d_out), jnp.float32)],
  )(agg_in, agg_out, ent_p, in_w, out_w, loop_w,
    loop_rel, bias_cov.reshape(1, d_out), rel_emb, w_rel)

  pair_gather = _make_pair_gather(d_out, batch)
  head = jnp.asarray(triples[:, 0], jnp.int32)
  rela = jnp.asarray(triples[:, 1], jnp.int32)
  xh, rh = pair_gather(xpre, r, head, rela)

  bblk = 128
  nbb = batch // bblk
  score = pl.pallas_call(
      functools.partial(_decoder_body, n_ent=float(n_ent), bblk=bblk),
      grid=(nbb,),
      in_specs=[
          pl.BlockSpec((batch, d_out), lambda i: (0, 0)),
          pl.BlockSpec((batch, d_out), lambda i: (0, 0)),
          pl.BlockSpec((8, d_out), lambda i: (0, 0)),
          pl.BlockSpec((1, d_out), lambda i: (0, 0)),
          pl.BlockSpec((1, d_out), lambda i: (0, 0)),
          pl.BlockSpec((n_ent, d_out), lambda i: (0, 0)),
          pl.BlockSpec((1, n_ent), lambda i: (0, 0)),
      ],
      out_specs=pl.BlockSpec((bblk, n_ent), lambda i: (i, 0)),
      out_shape=jax.ShapeDtypeStruct((batch, n_ent), jnp.float32),
      scratch_shapes=[pltpu.VMEM((batch, d_out), jnp.float32)],
  )(xh, rh, stats, bn_gamma.reshape(1, d_out), bn_beta.reshape(1, d_out),
    emb_ent, b_ent.reshape(1, n_ent))

  return score
